# baseline (speedup 1.0000x reference)
import jax
import jax.numpy as jnp
from jax.experimental import pallas as pl
from jax.experimental.pallas import tpu as pltpu


def kernel(x, embed_table):
    b, _, h, w = x.shape
    hw = h * w
    d = embed_table.shape[1]

    bb = 8
    grid = b // bb

    def body(e_ref, o_ref, scratch):
        @pl.when(pl.program_id(0) == 0)
        def _():
            scratch[...] = e_ref[...].T.reshape(d, h, w)

        o_ref[...] = jnp.broadcast_to(scratch[...][None], (bb, d, h, w))

    return pl.pallas_call(
        body,
        grid=(grid,),
        in_specs=[pl.BlockSpec((hw, d), lambda i: (0, 0))],
        out_specs=pl.BlockSpec((bb, d, h, w), lambda i: (i, 0, 0, 0)),
        out_shape=jax.ShapeDtypeStruct((b, d, h, w), embed_table.dtype),
        scratch_shapes=[pltpu.VMEM((d, h, w), embed_table.dtype)],
    )(embed_table)
